# baseline (speedup 1.0000x reference)
import functools

import jax
import jax.numpy as jnp
from jax import lax
from jax.experimental import pallas as pl
from jax.experimental.pallas import tpu as pltpu
from jax.experimental.pallas import tpu_sc as plsc

HIDDEN = 256
EDGE_HIDDEN = 16
N_NODES = 10000
N_EDGES = 160000

_NSC = 2
_NTILE = 16
_NW = _NSC * _NTILE
_WCOL = 128
_NPAD = 10240
_NSTRIPE = _NPAD // _NTILE

_E1 = 81920
_E2 = N_EDGES - _E1

_GCH = 128
_SCH = 80
_BE = 4000


def _sc_gather(node_i32, src, base, epw, n_edges):
    mesh = plsc.VectorSubcoreMesh(core_axis_name="c", subcore_axis_name="s")
    full = epw // _GCH
    tail = epw - full * _GCH

    @functools.partial(
        pl.kernel,
        out_type=jax.ShapeDtypeStruct((n_edges, HIDDEN // 2), jnp.int32),
        mesh=mesh,
        scratch_types=[
            pltpu.VMEM((epw,), jnp.int32),
            pltpu.VMEM((_GCH, HIDDEN // 2), jnp.int32),
            pltpu.VMEM((_GCH, HIDDEN // 2), jnp.int32),
            pltpu.VMEM((max(tail, 8), HIDDEN // 2), jnp.int32),
            pltpu.SemaphoreType.DMA,
            pltpu.SemaphoreType.DMA,
            pltpu.SemaphoreType.DMA,
            pltpu.SemaphoreType.DMA,
        ],
    )
    def k(table_hbm, src_hbm, out_hbm, idx_v, rows_a, rows_b, tail_v,
          sem_ga, sem_gb, sem_wa, sem_wb):
        c = lax.axis_index("c")
        s = lax.axis_index("s")
        wid = s * _NSC + c
        ob = wid * epw
        pltpu.sync_copy(src_hbm.at[pl.ds(base + ob, epw)], idx_v)

        def start_g(j, rows_ref, sem_g):
            pltpu.async_copy(
                table_hbm.at[idx_v.at[pl.ds(j * _GCH, _GCH)]], rows_ref, sem_g
            )

        def half(j, rows_ref, sem_g, sem_w):
            pltpu.make_async_copy(
                table_hbm.at[idx_v.at[pl.ds(0, _GCH)]], rows_ref, sem_g
            ).wait()
            pltpu.async_copy(
                rows_ref, out_hbm.at[pl.ds(ob + j * _GCH, _GCH)], sem_w
            )
            pltpu.make_async_copy(
                rows_ref, out_hbm.at[pl.ds(ob, _GCH)], sem_w
            ).wait()

            @pl.when(j + 2 <= full - 1)
            def _():
                start_g(j + 2, rows_ref, sem_g)

        start_g(0, rows_a, sem_ga)
        start_g(1, rows_b, sem_gb)

        def body(g, carry):
            half(2 * g, rows_a, sem_ga, sem_wa)
            half(2 * g + 1, rows_b, sem_gb, sem_wb)
            return carry

        lax.fori_loop(0, full // 2, body, 0)
        if full % 2 == 1:
            half(full - 1, rows_a, sem_ga, sem_wa)
        if tail:
            t0 = full * _GCH
            pltpu.async_copy(
                table_hbm.at[idx_v.at[pl.ds(t0, tail)]],
                tail_v.at[pl.ds(0, tail)], sem_ga
            ).wait()
            pltpu.sync_copy(tail_v.at[pl.ds(0, tail)],
                            out_hbm.at[pl.ds(ob + t0, tail)])

    return k(node_i32, src)


def _sc_scatter(msgs, dst, init):
    mesh = plsc.VectorSubcoreMesh(core_axis_name="c", subcore_axis_name="s")

    @functools.partial(
        pl.kernel,
        out_type=(
            jax.ShapeDtypeStruct((_NSC, _NPAD, _WCOL), jnp.float32),
            jax.ShapeDtypeStruct((_NPAD,), jnp.float32),
            jax.ShapeDtypeStruct((_NTILE, 1, _NPAD), jnp.float32),
        ),
        mesh=mesh,
        scratch_types=[
            pltpu.VMEM((_SCH,), jnp.int32),
            pltpu.VMEM((_SCH,), jnp.int32),
            pltpu.VMEM((_SCH, _WCOL), jnp.float32),
            pltpu.VMEM((_SCH, _WCOL), jnp.float32),
            pltpu.VMEM((1, _NPAD), jnp.float32),
            pltpu.VMEM((_NTILE, 1, _NSTRIPE), jnp.float32),
            pltpu.VMEM((_NSTRIPE,), jnp.float32),
            pltpu.VMEM_SHARED((_NPAD, _WCOL), jnp.float32),
            pltpu.SemaphoreType.DMA,
            pltpu.SemaphoreType.DMA,
            pltpu.SemaphoreType.DMA,
            pltpu.SemaphoreType.DMA,
        ],
        compiler_params=pltpu.CompilerParams(needs_layout_passes=False),
    )
    def k(msgs_hbm, dst_hbm, init_hbm, out_hbm, deg_hbm,
          stage_hbm, idx_a, idx_b, buf_a, buf_b, hist_v, red_v,
          res_v, acc_s, sem_ia, sem_ib, sem_ma, sem_mb):
        c = lax.axis_index("c")
        s = lax.axis_index("s")
        pltpu.sync_copy(init_hbm.at[c, pl.ds(s * _NSTRIPE, _NSTRIPE)],
                        acc_s.at[pl.ds(s * _NSTRIPE, _NSTRIPE)])

        @pl.when(c == 0)
        def _zero_hist():
            zv = jnp.zeros((16,), jnp.float32)

            def zbody(i, carry):
                hist_v[0, pl.ds(i * 16, 16)] = zv
                return carry

            lax.fori_loop(0, _NPAD // 16, zbody, 0)

        plsc.subcore_barrier()
        vones = jnp.full((16,), 1.0, jnp.float32)
        ept = N_EDGES // _NTILE
        srows = ept // _SCH
        e0 = s * ept

        def start(j, idx_ref, buf_ref, sem_i, sem_m):
            pltpu.async_copy(dst_hbm.at[pl.ds(e0 + j * _SCH, _SCH)],
                             idx_ref, sem_i)
            pltpu.async_copy(
                msgs_hbm.at[c, pl.ds(e0 + j * _SCH, _SCH)], buf_ref, sem_m)

        def fin(idx_ref, buf_ref, sem_i, sem_m):
            pltpu.make_async_copy(dst_hbm.at[pl.ds(e0, _SCH)], idx_ref,
                                  sem_i).wait()
            pltpu.make_async_copy(msgs_hbm.at[c, pl.ds(e0, _SCH)],
                                  buf_ref, sem_m).wait()
            pltpu.sync_copy(buf_ref, acc_s.at[idx_ref], add=True)

            @pl.when(c == 0)
            def _hist():
                for t in range(_SCH // 16):
                    idx16 = idx_ref[pl.ds(t * 16, 16)]
                    plsc.addupdate_scatter(hist_v.at[0], [idx16], vones)

        start(0, idx_a, buf_a, sem_ia, sem_ma)
        start(1, idx_b, buf_b, sem_ib, sem_mb)

        def body(g, carry):
            fin(idx_a, buf_a, sem_ia, sem_ma)

            @pl.when(2 * g + 2 <= srows - 1)
            def _():
                start(2 * g + 2, idx_a, buf_a, sem_ia, sem_ma)

            fin(idx_b, buf_b, sem_ib, sem_mb)

            @pl.when(2 * g + 3 <= srows - 1)
            def _():
                start(2 * g + 3, idx_b, buf_b, sem_ib, sem_mb)

            return carry

        lax.fori_loop(0, srows // 2, body, 0)
        if srows % 2 == 1:
            fin(idx_a, buf_a, sem_ia, sem_ma)

        plsc.subcore_barrier()
        pltpu.sync_copy(
            acc_s.at[pl.ds(s * _NSTRIPE, _NSTRIPE)],
            out_hbm.at[c, pl.ds(s * _NSTRIPE, _NSTRIPE)],
        )

        if True:
            @pl.when(c == 0)
            def _deg_reduce():
                pltpu.sync_copy(hist_v, stage_hbm.at[s])
                plsc.subcore_barrier()
                pltpu.sync_copy(
                    stage_hbm.at[:, :, pl.ds(s * _NSTRIPE, _NSTRIPE)], red_v
                )

                def rbody(g, carry):
                    v = red_v[0, 0, pl.ds(g * 16, 16)]
                    for t in range(1, _NTILE):
                        v = v + red_v[t, 0, pl.ds(g * 16, 16)]
                    res_v[pl.ds(g * 16, 16)] = v
                    return carry

                lax.fori_loop(0, _NSTRIPE // 16, rbody, 0)
                pltpu.sync_copy(res_v,
                                deg_hbm.at[pl.ds(s * _NSTRIPE, _NSTRIPE)])

    return k(msgs, dst, init)


def _msg_body(eb_ref, sf_ref, w1_ref, b1_ref, w2_ref, b2_ref, out_ref):
    h = jnp.maximum(
        jnp.dot(eb_ref[...], w1_ref[...], preferred_element_type=jnp.float32)
        + b1_ref[...],
        0.0,
    )
    ew = jnp.dot(h.astype(jnp.bfloat16), w2_ref[...],
                 preferred_element_type=jnp.float32) + b2_ref[...]
    x = sf_ref[...]
    sf_lo = lax.bitcast_convert_type(x << 16, jnp.float32)
    sf_hi = lax.bitcast_convert_type(x & jnp.int32(-65536), jnp.float32)
    out_ref[0] = ew[:, :_WCOL] * sf_lo
    out_ref[1] = ew[:, _WCOL:] * sf_hi


def _tc_messages(edge_emb, sf, ew_W1, ew_b1, ew_W2, ew_b2, blk_ofs, n_edges):
    grid = n_edges // _BE
    return pl.pallas_call(
        _msg_body,
        grid=(grid,),
        in_specs=[
            pl.BlockSpec((_BE, EDGE_HIDDEN), lambda i, o=blk_ofs: (i + o, 0)),
            pl.BlockSpec((_BE, HIDDEN // 2), lambda i: (i, 0)),
            pl.BlockSpec((EDGE_HIDDEN, HIDDEN), lambda i: (0, 0)),
            pl.BlockSpec((1, HIDDEN), lambda i: (0, 0)),
            pl.BlockSpec((HIDDEN, HIDDEN), lambda i: (0, 0)),
            pl.BlockSpec((1, HIDDEN), lambda i: (0, 0)),
        ],
        out_specs=pl.BlockSpec((_NSC, _BE, _WCOL), lambda i: (0, i, 0)),
        out_shape=jax.ShapeDtypeStruct((_NSC, n_edges, _WCOL), jnp.float32),
    )(edge_emb.astype(jnp.bfloat16), sf, ew_W1.astype(jnp.bfloat16),
      ew_b1.reshape(1, -1), ew_W2.astype(jnp.bfloat16),
      ew_b2.reshape(1, -1))


def _final_body(x_ref, agg_ref, deg_ref, w1a_ref, w1b_ref, b1_ref, w2_ref,
                b2_ref, g_ref, beta_ref, out_ref):
    x = x_ref[...]
    deg = jnp.maximum(deg_ref[...], 1.0)
    a = jnp.concatenate([agg_ref[0], agg_ref[1]], axis=1) / deg
    h2 = jnp.maximum(
        jnp.dot(x.astype(jnp.bfloat16), w1a_ref[...].astype(jnp.bfloat16),
                preferred_element_type=jnp.float32)
        + jnp.dot(a.astype(jnp.bfloat16), w1b_ref[...].astype(jnp.bfloat16),
                  preferred_element_type=jnp.float32)
        + b1_ref[...],
        0.0,
    )
    nu = jnp.dot(h2.astype(jnp.bfloat16), w2_ref[...].astype(jnp.bfloat16),
                 preferred_element_type=jnp.float32) + b2_ref[...]
    y = x + nu
    mean = jnp.mean(y, axis=1, keepdims=True)
    yc = y - mean
    var = jnp.mean(yc * yc, axis=1, keepdims=True)
    out_ref[...] = yc * lax.rsqrt(var + 1e-5) * g_ref[...] + beta_ref[...]


def _tc_final(node_emb, agg, deg, nu_W1, nu_b1, nu_W2, nu_b2, ln_gamma,
              ln_beta):
    BN = 1000
    grid = N_NODES // BN
    return pl.pallas_call(
        _final_body,
        grid=(grid,),
        in_specs=[
            pl.BlockSpec((BN, HIDDEN), lambda i: (i, 0)),
            pl.BlockSpec((_NSC, BN, _WCOL), lambda i: (0, i, 0)),
            pl.BlockSpec((BN, 1), lambda i: (i, 0)),
            pl.BlockSpec((HIDDEN, HIDDEN), lambda i: (0, 0)),
            pl.BlockSpec((HIDDEN, HIDDEN), lambda i: (0, 0)),
            pl.BlockSpec((1, HIDDEN), lambda i: (0, 0)),
            pl.BlockSpec((HIDDEN, HIDDEN), lambda i: (0, 0)),
            pl.BlockSpec((1, HIDDEN), lambda i: (0, 0)),
            pl.BlockSpec((1, HIDDEN), lambda i: (0, 0)),
            pl.BlockSpec((1, HIDDEN), lambda i: (0, 0)),
        ],
        out_specs=pl.BlockSpec((BN, HIDDEN), lambda i: (i, 0)),
        out_shape=jax.ShapeDtypeStruct((N_NODES, HIDDEN), jnp.float32),
    )(node_emb, agg, deg, nu_W1[:HIDDEN], nu_W1[HIDDEN:],
      nu_b1.reshape(1, -1), nu_W2, nu_b2.reshape(1, -1),
      ln_gamma.reshape(1, -1), ln_beta.reshape(1, -1))


def kernel(node_emb, edge_index, edge_emb, ew_W1, ew_b1, ew_W2, ew_b2,
           nu_W1, nu_b1, nu_W2, nu_b2, ln_gamma, ln_beta):
    src = edge_index[0].astype(jnp.int32)
    dst = edge_index[1].astype(jnp.int32)
    node_bf = node_emb.astype(jnp.bfloat16)
    lo = lax.bitcast_convert_type(node_bf[:, : HIDDEN // 2],
                                  jnp.uint16).astype(jnp.uint32)
    hi = lax.bitcast_convert_type(node_bf[:, HIDDEN // 2:],
                                  jnp.uint16).astype(jnp.uint32)
    node_i32 = lax.bitcast_convert_type(lo | (hi << 16), jnp.int32)

    sf = _sc_gather(node_i32, src, 0, N_EDGES // _NW, N_EDGES)
    m = _tc_messages(edge_emb, sf, ew_W1, ew_b1, ew_W2, ew_b2, 0, N_EDGES)
    zinit = jnp.zeros((_NSC, _NPAD, _WCOL), jnp.float32)
    agg, deg, _stage = _sc_scatter(m, dst, zinit)
    deg2d = deg.reshape(_NPAD, 1)
    return _tc_final(node_emb, agg, deg2d, nu_W1, nu_b1, nu_W2, nu_b2,
                     ln_gamma, ln_beta)


# speedup vs baseline: 1.0112x; 1.0112x over previous
import functools

import jax
import jax.numpy as jnp
from jax import lax
from jax.experimental import pallas as pl
from jax.experimental.pallas import tpu as pltpu
from jax.experimental.pallas import tpu_sc as plsc

HIDDEN = 256
EDGE_HIDDEN = 16
N_NODES = 10000
N_EDGES = 160000

_NSC = 2
_NTILE = 16
_NW = _NSC * _NTILE
_WCOL = 128
_NPAD = 10240
_NSTRIPE = _NPAD // _NTILE

_E1 = 81920
_E2 = N_EDGES - _E1

_GCH = 128
_SCH = 80
_BE = 8000


def _sc_gather(node_i32, src, base, epw, n_edges):
    mesh = plsc.VectorSubcoreMesh(core_axis_name="c", subcore_axis_name="s")
    full = epw // _GCH
    tail = epw - full * _GCH

    @functools.partial(
        pl.kernel,
        out_type=jax.ShapeDtypeStruct((n_edges, HIDDEN // 2), jnp.int32),
        mesh=mesh,
        scratch_types=[
            pltpu.VMEM((epw,), jnp.int32),
            pltpu.VMEM((_GCH, HIDDEN // 2), jnp.int32),
            pltpu.VMEM((_GCH, HIDDEN // 2), jnp.int32),
            pltpu.VMEM((max(tail, 8), HIDDEN // 2), jnp.int32),
            pltpu.SemaphoreType.DMA,
            pltpu.SemaphoreType.DMA,
            pltpu.SemaphoreType.DMA,
            pltpu.SemaphoreType.DMA,
        ],
    )
    def k(table_hbm, src_hbm, out_hbm, idx_v, rows_a, rows_b, tail_v,
          sem_ga, sem_gb, sem_wa, sem_wb):
        c = lax.axis_index("c")
        s = lax.axis_index("s")
        wid = s * _NSC + c
        ob = wid * epw
        pltpu.sync_copy(src_hbm.at[pl.ds(base + ob, epw)], idx_v)

        def start_g(j, rows_ref, sem_g):
            pltpu.async_copy(
                table_hbm.at[idx_v.at[pl.ds(j * _GCH, _GCH)]], rows_ref, sem_g
            )

        def half(j, rows_ref, sem_g, sem_w):
            pltpu.make_async_copy(
                table_hbm.at[idx_v.at[pl.ds(0, _GCH)]], rows_ref, sem_g
            ).wait()
            pltpu.async_copy(
                rows_ref, out_hbm.at[pl.ds(ob + j * _GCH, _GCH)], sem_w
            )
            pltpu.make_async_copy(
                rows_ref, out_hbm.at[pl.ds(ob, _GCH)], sem_w
            ).wait()

            @pl.when(j + 2 <= full - 1)
            def _():
                start_g(j + 2, rows_ref, sem_g)

        start_g(0, rows_a, sem_ga)
        start_g(1, rows_b, sem_gb)

        def body(g, carry):
            half(2 * g, rows_a, sem_ga, sem_wa)
            half(2 * g + 1, rows_b, sem_gb, sem_wb)
            return carry

        lax.fori_loop(0, full // 2, body, 0)
        if full % 2 == 1:
            half(full - 1, rows_a, sem_ga, sem_wa)
        if tail:
            t0 = full * _GCH
            pltpu.async_copy(
                table_hbm.at[idx_v.at[pl.ds(t0, tail)]],
                tail_v.at[pl.ds(0, tail)], sem_ga
            ).wait()
            pltpu.sync_copy(tail_v.at[pl.ds(0, tail)],
                            out_hbm.at[pl.ds(ob + t0, tail)])

    return k(node_i32, src)


def _sc_scatter(msgs, dst, init):
    mesh = plsc.VectorSubcoreMesh(core_axis_name="c", subcore_axis_name="s")

    @functools.partial(
        pl.kernel,
        out_type=(
            jax.ShapeDtypeStruct((_NSC, _NPAD, _WCOL), jnp.float32),
            jax.ShapeDtypeStruct((_NPAD,), jnp.float32),
            jax.ShapeDtypeStruct((_NTILE, 1, _NPAD), jnp.float32),
        ),
        mesh=mesh,
        scratch_types=[
            pltpu.VMEM((_SCH,), jnp.int32),
            pltpu.VMEM((_SCH,), jnp.int32),
            pltpu.VMEM((_SCH, _WCOL), jnp.float32),
            pltpu.VMEM((_SCH, _WCOL), jnp.float32),
            pltpu.VMEM((1, _NPAD), jnp.float32),
            pltpu.VMEM((_NTILE, 1, _NSTRIPE), jnp.float32),
            pltpu.VMEM((_NSTRIPE,), jnp.float32),
            pltpu.VMEM_SHARED((_NPAD, _WCOL), jnp.float32),
            pltpu.SemaphoreType.DMA,
            pltpu.SemaphoreType.DMA,
            pltpu.SemaphoreType.DMA,
            pltpu.SemaphoreType.DMA,
        ],
        compiler_params=pltpu.CompilerParams(needs_layout_passes=False),
    )
    def k(msgs_hbm, dst_hbm, init_hbm, out_hbm, deg_hbm,
          stage_hbm, idx_a, idx_b, buf_a, buf_b, hist_v, red_v,
          res_v, acc_s, sem_ia, sem_ib, sem_ma, sem_mb):
        c = lax.axis_index("c")
        s = lax.axis_index("s")
        pltpu.sync_copy(init_hbm.at[c, pl.ds(s * _NSTRIPE, _NSTRIPE)],
                        acc_s.at[pl.ds(s * _NSTRIPE, _NSTRIPE)])

        @pl.when(c == 0)
        def _zero_hist():
            zv = jnp.zeros((16,), jnp.float32)

            def zbody(i, carry):
                hist_v[0, pl.ds(i * 16, 16)] = zv
                return carry

            lax.fori_loop(0, _NPAD // 16, zbody, 0)

        plsc.subcore_barrier()
        vones = jnp.full((16,), 1.0, jnp.float32)
        ept = N_EDGES // _NTILE
        srows = ept // _SCH
        e0 = s * ept

        def start(j, idx_ref, buf_ref, sem_i, sem_m):
            pltpu.async_copy(dst_hbm.at[pl.ds(e0 + j * _SCH, _SCH)],
                             idx_ref, sem_i)
            pltpu.async_copy(
                msgs_hbm.at[c, pl.ds(e0 + j * _SCH, _SCH)], buf_ref, sem_m)

        def fin(idx_ref, buf_ref, sem_i, sem_m):
            pltpu.make_async_copy(dst_hbm.at[pl.ds(e0, _SCH)], idx_ref,
                                  sem_i).wait()
            pltpu.make_async_copy(msgs_hbm.at[c, pl.ds(e0, _SCH)],
                                  buf_ref, sem_m).wait()
            pltpu.sync_copy(buf_ref, acc_s.at[idx_ref], add=True)

            @pl.when(c == 0)
            def _hist():
                for t in range(_SCH // 16):
                    idx16 = idx_ref[pl.ds(t * 16, 16)]
                    plsc.addupdate_scatter(hist_v.at[0], [idx16], vones)

        start(0, idx_a, buf_a, sem_ia, sem_ma)
        start(1, idx_b, buf_b, sem_ib, sem_mb)

        def body(g, carry):
            fin(idx_a, buf_a, sem_ia, sem_ma)

            @pl.when(2 * g + 2 <= srows - 1)
            def _():
                start(2 * g + 2, idx_a, buf_a, sem_ia, sem_ma)

            fin(idx_b, buf_b, sem_ib, sem_mb)

            @pl.when(2 * g + 3 <= srows - 1)
            def _():
                start(2 * g + 3, idx_b, buf_b, sem_ib, sem_mb)

            return carry

        lax.fori_loop(0, srows // 2, body, 0)
        if srows % 2 == 1:
            fin(idx_a, buf_a, sem_ia, sem_ma)

        plsc.subcore_barrier()
        pltpu.sync_copy(
            acc_s.at[pl.ds(s * _NSTRIPE, _NSTRIPE)],
            out_hbm.at[c, pl.ds(s * _NSTRIPE, _NSTRIPE)],
        )

        if True:
            @pl.when(c == 0)
            def _deg_reduce():
                pltpu.sync_copy(hist_v, stage_hbm.at[s])
                plsc.subcore_barrier()
                pltpu.sync_copy(
                    stage_hbm.at[:, :, pl.ds(s * _NSTRIPE, _NSTRIPE)], red_v
                )

                def rbody(g, carry):
                    v = red_v[0, 0, pl.ds(g * 16, 16)]
                    for t in range(1, _NTILE):
                        v = v + red_v[t, 0, pl.ds(g * 16, 16)]
                    res_v[pl.ds(g * 16, 16)] = v
                    return carry

                lax.fori_loop(0, _NSTRIPE // 16, rbody, 0)
                pltpu.sync_copy(res_v,
                                deg_hbm.at[pl.ds(s * _NSTRIPE, _NSTRIPE)])

    return k(msgs, dst, init)


def _msg_body(eb_ref, sf_ref, w1_ref, b1_ref, w2_ref, b2_ref, out_ref):
    h = jnp.maximum(
        jnp.dot(eb_ref[...], w1_ref[...], preferred_element_type=jnp.float32)
        + b1_ref[...],
        0.0,
    )
    ew = jnp.dot(h.astype(jnp.bfloat16), w2_ref[...],
                 preferred_element_type=jnp.float32) + b2_ref[...]
    x = sf_ref[...]
    sf_lo = lax.bitcast_convert_type(x << 16, jnp.float32)
    sf_hi = lax.bitcast_convert_type(x & jnp.int32(-65536), jnp.float32)
    out_ref[0] = ew[:, :_WCOL] * sf_lo
    out_ref[1] = ew[:, _WCOL:] * sf_hi


def _tc_messages(edge_emb, sf, ew_W1, ew_b1, ew_W2, ew_b2, blk_ofs, n_edges):
    grid = n_edges // _BE
    return pl.pallas_call(
        _msg_body,
        grid=(grid,),
        in_specs=[
            pl.BlockSpec((_BE, EDGE_HIDDEN), lambda i, o=blk_ofs: (i + o, 0)),
            pl.BlockSpec((_BE, HIDDEN // 2), lambda i: (i, 0)),
            pl.BlockSpec((EDGE_HIDDEN, HIDDEN), lambda i: (0, 0)),
            pl.BlockSpec((1, HIDDEN), lambda i: (0, 0)),
            pl.BlockSpec((HIDDEN, HIDDEN), lambda i: (0, 0)),
            pl.BlockSpec((1, HIDDEN), lambda i: (0, 0)),
        ],
        out_specs=pl.BlockSpec((_NSC, _BE, _WCOL), lambda i: (0, i, 0)),
        out_shape=jax.ShapeDtypeStruct((_NSC, n_edges, _WCOL), jnp.float32),
    )(edge_emb.astype(jnp.bfloat16), sf, ew_W1.astype(jnp.bfloat16),
      ew_b1.reshape(1, -1), ew_W2.astype(jnp.bfloat16),
      ew_b2.reshape(1, -1))


def _final_body(x_ref, agg_ref, deg_ref, w1a_ref, w1b_ref, b1_ref, w2_ref,
                b2_ref, g_ref, beta_ref, out_ref):
    x = x_ref[...]
    deg = jnp.maximum(deg_ref[...], 1.0)
    a = jnp.concatenate([agg_ref[0], agg_ref[1]], axis=1) / deg
    h2 = jnp.maximum(
        jnp.dot(x.astype(jnp.bfloat16), w1a_ref[...].astype(jnp.bfloat16),
                preferred_element_type=jnp.float32)
        + jnp.dot(a.astype(jnp.bfloat16), w1b_ref[...].astype(jnp.bfloat16),
                  preferred_element_type=jnp.float32)
        + b1_ref[...],
        0.0,
    )
    nu = jnp.dot(h2.astype(jnp.bfloat16), w2_ref[...].astype(jnp.bfloat16),
                 preferred_element_type=jnp.float32) + b2_ref[...]
    y = x + nu
    mean = jnp.mean(y, axis=1, keepdims=True)
    yc = y - mean
    var = jnp.mean(yc * yc, axis=1, keepdims=True)
    out_ref[...] = yc * lax.rsqrt(var + 1e-5) * g_ref[...] + beta_ref[...]


def _tc_final(node_emb, agg, deg, nu_W1, nu_b1, nu_W2, nu_b2, ln_gamma,
              ln_beta):
    BN = 1000
    grid = N_NODES // BN
    return pl.pallas_call(
        _final_body,
        grid=(grid,),
        in_specs=[
            pl.BlockSpec((BN, HIDDEN), lambda i: (i, 0)),
            pl.BlockSpec((_NSC, BN, _WCOL), lambda i: (0, i, 0)),
            pl.BlockSpec((BN, 1), lambda i: (i, 0)),
            pl.BlockSpec((HIDDEN, HIDDEN), lambda i: (0, 0)),
            pl.BlockSpec((HIDDEN, HIDDEN), lambda i: (0, 0)),
            pl.BlockSpec((1, HIDDEN), lambda i: (0, 0)),
            pl.BlockSpec((HIDDEN, HIDDEN), lambda i: (0, 0)),
            pl.BlockSpec((1, HIDDEN), lambda i: (0, 0)),
            pl.BlockSpec((1, HIDDEN), lambda i: (0, 0)),
            pl.BlockSpec((1, HIDDEN), lambda i: (0, 0)),
        ],
        out_specs=pl.BlockSpec((BN, HIDDEN), lambda i: (i, 0)),
        out_shape=jax.ShapeDtypeStruct((N_NODES, HIDDEN), jnp.float32),
    )(node_emb, agg, deg, nu_W1[:HIDDEN], nu_W1[HIDDEN:],
      nu_b1.reshape(1, -1), nu_W2, nu_b2.reshape(1, -1),
      ln_gamma.reshape(1, -1), ln_beta.reshape(1, -1))


def kernel(node_emb, edge_index, edge_emb, ew_W1, ew_b1, ew_W2, ew_b2,
           nu_W1, nu_b1, nu_W2, nu_b2, ln_gamma, ln_beta):
    src = edge_index[0].astype(jnp.int32)
    dst = edge_index[1].astype(jnp.int32)
    node_bf = node_emb.astype(jnp.bfloat16)
    lo = lax.bitcast_convert_type(node_bf[:, : HIDDEN // 2],
                                  jnp.uint16).astype(jnp.uint32)
    hi = lax.bitcast_convert_type(node_bf[:, HIDDEN // 2:],
                                  jnp.uint16).astype(jnp.uint32)
    node_i32 = lax.bitcast_convert_type(lo | (hi << 16), jnp.int32)

    sf = _sc_gather(node_i32, src, 0, N_EDGES // _NW, N_EDGES)
    m = _tc_messages(edge_emb, sf, ew_W1, ew_b1, ew_W2, ew_b2, 0, N_EDGES)
    zinit = jnp.zeros((_NSC, _NPAD, _WCOL), jnp.float32)
    agg, deg, _stage = _sc_scatter(m, dst, zinit)
    deg2d = deg.reshape(_NPAD, 1)
    return _tc_final(node_emb, agg, deg2d, nu_W1, nu_b1, nu_W2, nu_b2,
                     ln_gamma, ln_beta)
